# baseline (speedup 1.0000x reference)
import jax
import jax.numpy as jnp
from jax import lax
from jax.experimental import pallas as pl
from jax.experimental.pallas import tpu as pltpu
from jax.experimental.pallas import tpu_sc as plsc

B, T, D, E = 2, 2048, 1024, 8
N = B * T
NC, NS, L = 2, 16, 16
NW = NC * NS
TPW = N // NW
DE = D * E


def _sc_body(experts_hbm, gate_hbm, out_hbm, gate_v, eidx_v, slab_v, row_v,
             gsem, isem, osem):
    wid = lax.axis_index("s") * NC + lax.axis_index("c")
    base = wid * TPW
    pltpu.async_copy(gate_hbm.at[pl.ds(base * E, TPW * E)], gate_v, gsem).wait()

    lane = lax.iota(jnp.int32, L)

    @pl.loop(0, TPW // L)
    def _(c):
        tok0 = lane * E + c * (L * E)
        bv = plsc.load_gather(gate_v, [tok0])
        bi = jnp.zeros((L,), jnp.int32)
        for e in range(1, E):
            v = plsc.load_gather(gate_v, [tok0 + e])
            better = v > bv
            bv = jnp.where(better, v, bv)
            bi = jnp.where(better, jnp.full((L,), e, jnp.int32), bi)
        eidx_v[pl.ds(c * L, L)] = bi

    pltpu.async_copy(experts_hbm.at[base], slab_v.at[0], isem)
    pltpu.async_copy(experts_hbm.at[base + 1], slab_v.at[1], isem)

    @pl.loop(0, TPW, step=2)
    def _(t0):
        for b in range(2):
            t = t0 + b
            pltpu.make_async_copy(experts_hbm.at[base], slab_v.at[b],
                                  isem).wait()
            @pl.when(t0 >= 2)
            def _():
                pltpu.make_async_copy(out_hbm.at[pl.ds(0, D)], row_v.at[b],
                                      osem).wait()

            e_splat = plsc.load_gather(eidx_v, [jnp.full((L,), t, jnp.int32)])
            slab = slab_v.at[b]
            row = row_v.at[b]

            @pl.loop(0, D // L)
            def _(c):
                idx = (lane + c * L) * E + e_splat
                row[pl.ds(c * L, L)] = plsc.load_gather(slab, [idx])

            @pl.when(t0 + b + 2 < TPW)
            def _():
                pltpu.async_copy(experts_hbm.at[base + t + 2], slab_v.at[b],
                                 isem)

            pltpu.async_copy(row_v.at[b], out_hbm.at[pl.ds((base + t) * D, D)],
                             osem)

    for b in range(2):
        pltpu.make_async_copy(out_hbm.at[pl.ds(0, D)], row_v.at[b],
                              osem).wait()


@jax.jit
def _run_sc(experts, gate):
    ef = experts.reshape(N, DE)
    gf = gate.reshape(N * E)
    mesh = plsc.VectorSubcoreMesh(core_axis_name="c", subcore_axis_name="s")
    out = pl.kernel(
        _sc_body,
        out_type=jax.ShapeDtypeStruct((N * D,), jnp.float32),
        mesh=mesh,
        compiler_params=pltpu.CompilerParams(use_tc_tiling_on_sc=False,
                                             needs_layout_passes=False),
        scratch_types=[
            pltpu.VMEM((TPW * E,), jnp.float32),
            pltpu.VMEM((TPW,), jnp.int32),
            pltpu.VMEM((2, DE), jnp.float32),
            pltpu.VMEM((2, D), jnp.float32),
            pltpu.SemaphoreType.DMA,
            pltpu.SemaphoreType.DMA,
            pltpu.SemaphoreType.DMA,
        ],
    )(ef, gf)
    return out.reshape(B, T, D)


TB = 64
R = DE // D
RW = D


def _amax_body(gate_ref, o_ref):
    g = gate_ref[...]
    amax = jnp.argmax(g, axis=-1).astype(jnp.int32)
    o_ref[...] = jnp.broadcast_to(amax[:, None], g.shape)


def _tc_body(amax_ref, s_ref, x_ref, o_ref):
    amax_r = amax_ref[...]
    lane_e = jax.lax.broadcasted_iota(jnp.int32, (TB * R, RW), 1) & (E - 1)
    x = x_ref[...]
    y = jnp.where(lane_e == amax_r, x, 0.0)
    hi = y.astype(jnp.bfloat16)
    lo = (y - hi.astype(jnp.float32)).astype(jnp.bfloat16)
    s = s_ref[...]
    dn = (((1,), (0,)), ((), ()))
    o_ref[...] = (
        jax.lax.dot_general(hi, s, dn, preferred_element_type=jnp.float32)
        + jax.lax.dot_general(lo, s, dn, preferred_element_type=jnp.float32)
    )


@jax.jit
def _run_tc(experts, gate):
    ef = experts.reshape(N * R, RW)
    gf = gate.reshape(N, E)
    amax8 = pl.pallas_call(
        _amax_body,
        out_shape=jax.ShapeDtypeStruct((N, E), jnp.int32),
    )(gf).reshape(N * R, 1)
    sel = (jnp.arange(RW, dtype=jnp.int32)[:, None] // E
           == jnp.arange(RW // E, dtype=jnp.int32)[None, :]
           ).astype(jnp.bfloat16)
    out = pl.pallas_call(
        _tc_body,
        grid=(N // TB,),
        in_specs=[
            pl.BlockSpec((TB * R, 1), lambda i: (i, 0)),
            pl.BlockSpec((RW, RW // E), lambda i: (0, 0)),
            pl.BlockSpec((TB * R, RW), lambda i: (i, 0)),
        ],
        out_specs=pl.BlockSpec((TB * R, RW // E), lambda i: (i, 0)),
        out_shape=jax.ShapeDtypeStruct((N * R, RW // E), jnp.float32),
    )(amax8, sel, ef)
    return out.reshape(B, T, D)


def kernel(experts, gate):
    return _run_tc(experts, gate)


# speedup vs baseline: 3.4653x; 3.4653x over previous
import jax
import jax.numpy as jnp
from jax import lax
from jax.experimental import pallas as pl
from jax.experimental.pallas import tpu as pltpu
from jax.experimental.pallas import tpu_sc as plsc

B, T, D, E = 2, 2048, 1024, 8
N = B * T
NC, NS, L = 2, 16, 16
NW = NC * NS
TPW = N // NW


def _sc_body(experts_hbm, gate_hbm, out_hbm, gate_v, eidx_v, gsem, dsem):
    wid = lax.axis_index("s") * NC + lax.axis_index("c")
    base = wid * TPW

    pltpu.async_copy(gate_hbm.at[:, pl.ds(base, TPW)], gate_v, gsem).wait()

    @pl.loop(0, TPW // L)
    def _(c):
        bv = gate_v[0, pl.ds(c * L, L)]
        bi = jnp.zeros((L,), jnp.int32)
        for e in range(1, E):
            v = gate_v[e, pl.ds(c * L, L)]
            better = v > bv
            bv = jnp.where(better, v, bv)
            bi = jnp.where(better, jnp.full((L,), e, jnp.int32), bi)
        eidx_v[pl.ds(c * L, L)] = bi

    @pl.loop(0, TPW)
    def _(i):
        t = base + i
        e_vec = plsc.load_gather(eidx_v, [jnp.full((L,), i, jnp.int32)])
        e = lax.reduce_max(e_vec, axes=(0,))
        pltpu.async_copy(experts_hbm.at[t, e, :], out_hbm.at[t], dsem)

    pltpu.make_async_copy(out_hbm.at[pl.ds(base, TPW)],
                          out_hbm.at[pl.ds(base, TPW)], dsem).wait()


@jax.jit
def _run(experts, gate):
    xt = jnp.swapaxes(experts, -1, -2).reshape(N, E, D)
    gt = jnp.swapaxes(gate.reshape(N, E), 0, 1)
    mesh = plsc.VectorSubcoreMesh(core_axis_name="c", subcore_axis_name="s")
    out = pl.kernel(
        _sc_body,
        out_type=jax.ShapeDtypeStruct((N, D), jnp.float32),
        mesh=mesh,
        compiler_params=pltpu.CompilerParams(needs_layout_passes=False),
        scratch_types=[
            pltpu.VMEM((E, TPW), jnp.float32),
            pltpu.VMEM((TPW,), jnp.int32),
            pltpu.SemaphoreType.DMA,
            pltpu.SemaphoreType.DMA,
        ],
    )(xt, gt)
    return out.reshape(B, T, D)


def kernel(experts, gate):
    return _run(experts, gate)


# speedup vs baseline: 54.5615x; 15.7450x over previous
import jax
import jax.numpy as jnp
from jax import lax
from jax.experimental import pallas as pl
from jax.experimental.pallas import tpu as pltpu
from jax.experimental.pallas import tpu_sc as plsc

B, T, D, E = 2, 2048, 1024, 8
N = B * T
NC, NS, L = 2, 16, 16
NW = NC * NS
TPW = N // NW


CH = 32
NCHUNK = TPW // CH


def _sc_body(experts_hbm, gate_hbm, out_hbm, gate_v, ridx_v, stage0_v,
             stage1_v, gsem, wsem0, wsem1):
    wid = lax.axis_index("s") * NC + lax.axis_index("c")
    base = wid * TPW

    pltpu.async_copy(gate_hbm.at[:, pl.ds(base, TPW)], gate_v, gsem).wait()

    lane = lax.iota(jnp.int32, L)

    @pl.loop(0, TPW // L)
    def _(c):
        bv = gate_v[0, pl.ds(c * L, L)]
        bi = jnp.zeros((L,), jnp.int32)
        for e in range(1, E):
            v = gate_v[e, pl.ds(c * L, L)]
            better = v > bv
            bv = jnp.where(better, v, bv)
            bi = jnp.where(better, jnp.full((L,), e, jnp.int32), bi)
        ridx_v[pl.ds(c * L, L)] = (base + c * L + lane) * E + bi

    bufs = (stage0_v, stage1_v)
    wsems = (wsem0, wsem1)
    pend_w = [None, None]

    def gather(i, buf):
        return pltpu.async_copy(
            experts_hbm.at[ridx_v.at[pl.ds(i * CH, CH)]], buf, gsem)

    gh = gather(0, bufs[0])
    for i in range(NCHUNK):
        b = i & 1
        gh.wait()
        if i + 1 < NCHUNK:
            nb = (i + 1) & 1
            if pend_w[nb] is not None:
                pend_w[nb].wait()
                pend_w[nb] = None
            gh = gather(i + 1, bufs[nb])
        pend_w[b] = pltpu.async_copy(
            bufs[b], out_hbm.at[pl.ds(base + i * CH, CH)], wsems[b])
    for b in range(2):
        if pend_w[b] is not None:
            pend_w[b].wait()


@jax.jit
def _run(experts, gate):
    xt = jnp.swapaxes(experts, -1, -2).reshape(N * E, D)
    gt = jnp.swapaxes(gate.reshape(N, E), 0, 1)
    mesh = plsc.VectorSubcoreMesh(core_axis_name="c", subcore_axis_name="s")
    out = pl.kernel(
        _sc_body,
        out_type=jax.ShapeDtypeStruct((N, D), jnp.float32),
        mesh=mesh,
        compiler_params=pltpu.CompilerParams(needs_layout_passes=False),
        scratch_types=[
            pltpu.VMEM((E, TPW), jnp.float32),
            pltpu.VMEM((TPW,), jnp.int32),
            pltpu.VMEM((CH, D), jnp.float32),
            pltpu.VMEM((CH, D), jnp.float32),
            pltpu.SemaphoreType.DMA,
            pltpu.SemaphoreType.DMA,
            pltpu.SemaphoreType.DMA,
        ],
    )(xt, gt)
    return out.reshape(B, T, D)


def kernel(experts, gate):
    return _run(experts, gate)


# speedup vs baseline: 55.5509x; 1.0181x over previous
import jax
import jax.numpy as jnp
from jax import lax
from jax.experimental import pallas as pl
from jax.experimental.pallas import tpu as pltpu
from jax.experimental.pallas import tpu_sc as plsc

B, T, D, E = 2, 2048, 1024, 8
N = B * T
NC, NS, L = 2, 16, 16
NW = NC * NS
TPW = N // NW


CH = 32
NCHUNK = TPW // CH


def _sc_body(experts_hbm, gate_hbm, out_hbm, gate_v, ridx_v, stage0_v,
             stage1_v, gsem0, gsem1, wsem0, wsem1):
    wid = lax.axis_index("s") * NC + lax.axis_index("c")
    base = wid * TPW

    pltpu.async_copy(gate_hbm.at[:, pl.ds(base, TPW)], gate_v, gsem0).wait()

    lane = lax.iota(jnp.int32, L)

    @pl.loop(0, TPW // L)
    def _(c):
        bv = gate_v[0, pl.ds(c * L, L)]
        bi = jnp.zeros((L,), jnp.int32)
        for e in range(1, E):
            v = gate_v[e, pl.ds(c * L, L)]
            better = v > bv
            bv = jnp.where(better, v, bv)
            bi = jnp.where(better, jnp.full((L,), e, jnp.int32), bi)
        ridx_v[pl.ds(c * L, L)] = (base + c * L + lane) * E + bi

    def gather(i, buf, sem):
        return pltpu.async_copy(
            experts_hbm.at[ridx_v.at[pl.ds(i * CH, CH)]], buf, sem)

    def write(i, buf, sem):
        return pltpu.async_copy(
            buf, out_hbm.at[pl.ds(base + i * CH, CH)], sem)

    g0 = gather(0, stage0_v, gsem0)
    g1 = gather(1, stage1_v, gsem1)
    g0.wait()
    w0 = write(0, stage0_v, wsem0)
    g1.wait()
    w1 = write(1, stage1_v, wsem1)
    w0.wait()
    g0 = gather(2, stage0_v, gsem0)
    w1.wait()
    g1 = gather(3, stage1_v, gsem1)
    g0.wait()
    w0 = write(2, stage0_v, wsem0)
    g1.wait()
    w1 = write(3, stage1_v, wsem1)
    w0.wait()
    w1.wait()


@jax.jit
def _run(experts, gate):
    xt = jnp.swapaxes(experts, -1, -2).reshape(N * E, D)
    gt = jnp.swapaxes(gate.reshape(N, E), 0, 1)
    mesh = plsc.VectorSubcoreMesh(core_axis_name="c", subcore_axis_name="s")
    out = pl.kernel(
        _sc_body,
        out_type=jax.ShapeDtypeStruct((N, D), jnp.float32),
        mesh=mesh,
        compiler_params=pltpu.CompilerParams(needs_layout_passes=False),
        scratch_types=[
            pltpu.VMEM((E, TPW), jnp.float32),
            pltpu.VMEM((TPW,), jnp.int32),
            pltpu.VMEM((CH, D), jnp.float32),
            pltpu.VMEM((CH, D), jnp.float32),
            pltpu.SemaphoreType.DMA,
            pltpu.SemaphoreType.DMA,
            pltpu.SemaphoreType.DMA,
            pltpu.SemaphoreType.DMA,
        ],
    )(xt, gt)
    return out.reshape(B, T, D)


def kernel(experts, gate):
    return _run(experts, gate)


# speedup vs baseline: 56.5666x; 1.0183x over previous
import jax
import jax.numpy as jnp
from jax import lax
from jax.experimental import pallas as pl
from jax.experimental.pallas import tpu as pltpu
from jax.experimental.pallas import tpu_sc as plsc

B, T, D, E = 2, 2048, 1024, 8
N = B * T
NC, NS, L = 2, 16, 16
NW = NC * NS
TPW = N // NW


HALF = TPW // 2


def _sc_body(experts_hbm, gate_hbm, out_hbm, gate_v, ridx_v, stage_v, gsem,
             dsem):
    wid = lax.axis_index("s") * NC + lax.axis_index("c")
    base = wid * TPW

    pltpu.async_copy(gate_hbm.at[:, pl.ds(base, TPW)], gate_v, gsem).wait()

    lane = lax.iota(jnp.int32, L)

    @pl.loop(0, TPW // L)
    def _(c):
        bv = gate_v[0, pl.ds(c * L, L)]
        bi = jnp.zeros((L,), jnp.int32)
        for e in range(1, E):
            v = gate_v[e, pl.ds(c * L, L)]
            better = v > bv
            bv = jnp.where(better, v, bv)
            bi = jnp.where(better, jnp.full((L,), e, jnp.int32), bi)
        ridx_v[pl.ds(c * L, L)] = (base + c * L + lane) * E + bi

    for h in range(2):
        pltpu.async_copy(experts_hbm.at[ridx_v.at[pl.ds(h * HALF, HALF)]],
                         stage_v, gsem).wait()
        pltpu.async_copy(stage_v, out_hbm.at[pl.ds(base + h * HALF, HALF)],
                         dsem).wait()


@jax.jit
def _run(experts, gate):
    xt = jnp.swapaxes(experts, -1, -2).reshape(N * E, D)
    gt = jnp.swapaxes(gate.reshape(N, E), 0, 1)
    mesh = plsc.VectorSubcoreMesh(core_axis_name="c", subcore_axis_name="s")
    out = pl.kernel(
        _sc_body,
        out_type=jax.ShapeDtypeStruct((N, D), jnp.float32),
        mesh=mesh,
        compiler_params=pltpu.CompilerParams(needs_layout_passes=False),
        scratch_types=[
            pltpu.VMEM((E, TPW), jnp.float32),
            pltpu.VMEM((TPW,), jnp.int32),
            pltpu.VMEM((HALF, D), jnp.float32),
            pltpu.SemaphoreType.DMA,
            pltpu.SemaphoreType.DMA,
        ],
    )(xt, gt)
    return out.reshape(B, T, D)


def kernel(experts, gate):
    return _run(experts, gate)
